# Initial kernel scaffold; baseline (speedup 1.0000x reference)
import jax
import jax.numpy as jnp
from jax.experimental import pallas as pl


def kernel(x, edge_index, W_self1, W_neigh1, W_self2, W_neigh2):
    raise NotImplementedError("write your pallas kernel here")


# speedup vs baseline: 4.7632x; 4.7632x over previous
import functools

import jax
import jax.numpy as jnp
from jax import lax
from jax.experimental import pallas as pl
from jax.experimental.pallas import tpu as pltpu
from jax.experimental.pallas import tpu_sc as plsc

N_NODES = 10000
N_EDGES = 320000
D = 128

NC = 2
NS = 16
NW = NC * NS
EDGES_PER_WORKER = N_EDGES // NW
CHUNK = 80
NUM_CHUNKS = EDGES_PER_WORKER // CHUNK
N_PAD = 10240
ROWS_PER_TILE = N_PAD // NS
DW = 16

_MESH = plsc.VectorSubcoreMesh(core_axis_name="c", subcore_axis_name="s")


@functools.partial(
    pl.kernel,
    mesh=_MESH,
    out_type=jax.ShapeDtypeStruct((NC, N_PAD, D), jnp.float32),
    scratch_types=(
        pltpu.VMEM((CHUNK,), jnp.int32),
        pltpu.VMEM((CHUNK, D), jnp.float32),
        pltpu.VMEM_SHARED((N_PAD, D), jnp.float32),
    ),
)
def _sc_deg(dst_hbm, zeros_hbm, ones_hbm, deg_out, dst_v, ones_v, deg_t):
    c = lax.axis_index("c")
    s = lax.axis_index("s")
    w = c * NS + s
    row0 = s * ROWS_PER_TILE

    pltpu.sync_copy(zeros_hbm, deg_t.at[pl.ds(row0, ROWS_PER_TILE)])
    pltpu.sync_copy(ones_hbm, ones_v)
    plsc.subcore_barrier()

    base = w * EDGES_PER_WORKER

    def body(i, carry):
        pltpu.sync_copy(dst_hbm.at[pl.ds(base + i * CHUNK, CHUNK)], dst_v)
        pltpu.sync_copy(ones_v, deg_t.at[dst_v], add=True)
        return carry

    lax.fori_loop(0, NUM_CHUNKS, body, 0)
    plsc.subcore_barrier()
    pltpu.sync_copy(
        deg_t.at[pl.ds(row0, ROWS_PER_TILE)],
        deg_out.at[c, pl.ds(row0, ROWS_PER_TILE)],
    )


@functools.partial(
    pl.kernel,
    mesh=_MESH,
    out_type=jax.ShapeDtypeStruct((NC, N_PAD, D), jnp.float32),
    scratch_types=(
        pltpu.VMEM((CHUNK,), jnp.int32),
        pltpu.VMEM((CHUNK,), jnp.int32),
        pltpu.VMEM((CHUNK, D), jnp.float32),
        pltpu.VMEM_SHARED((N_PAD, D), jnp.float32),
        pltpu.SemaphoreType.DMA,
    ),
)
def _sc_agg(x_hbm, src_hbm, dst_hbm, zeros_hbm, part_hbm, src_v, dst_v, rows_v, acc, sem):
    c = lax.axis_index("c")
    s = lax.axis_index("s")
    w = c * NS + s
    row0 = s * ROWS_PER_TILE

    pltpu.sync_copy(zeros_hbm, acc.at[pl.ds(row0, ROWS_PER_TILE)])
    plsc.subcore_barrier()

    base = w * EDGES_PER_WORKER

    def body(i, carry):
        off = base + i * CHUNK
        pltpu.sync_copy(src_hbm.at[pl.ds(off, CHUNK)], src_v)
        pltpu.sync_copy(dst_hbm.at[pl.ds(off, CHUNK)], dst_v)
        pltpu.async_copy(x_hbm.at[src_v], rows_v, sem).wait()
        pltpu.sync_copy(rows_v, acc.at[dst_v], add=True)
        return carry

    lax.fori_loop(0, NUM_CHUNKS, body, 0)
    plsc.subcore_barrier()

    pltpu.sync_copy(
        acc.at[pl.ds(row0, ROWS_PER_TILE)],
        part_hbm.at[c, pl.ds(row0, ROWS_PER_TILE)],
    )


BLK = 1000


def _combine_body(relu):
    def body(h_ref, p_ref, deg_ref, ws_ref, wn_ref, o_ref):
        degs = deg_ref[...]
        deg = degs[0, :, 0] + degs[1, :, 0]
        invd = 1.0 / jnp.maximum(deg, 1.0)
        agg = (p_ref[0] + p_ref[1]) * invd[:, None]
        out = jnp.dot(h_ref[...], ws_ref[...], preferred_element_type=jnp.float32)
        out = out + jnp.dot(agg, wn_ref[...], preferred_element_type=jnp.float32)
        if relu:
            out = jnp.maximum(out, 0.0)
        o_ref[...] = out

    return body


def _combine(h, parts, deg, w_self, w_neigh, relu):
    return pl.pallas_call(
        _combine_body(relu),
        grid=(N_NODES // BLK,),
        in_specs=[
            pl.BlockSpec((BLK, D), lambda i: (i, 0)),
            pl.BlockSpec((NC, BLK, D), lambda i: (0, i, 0)),
            pl.BlockSpec((NC, BLK, DW), lambda i: (0, i, 0)),
            pl.BlockSpec((D, D), lambda i: (0, 0)),
            pl.BlockSpec((D, D), lambda i: (0, 0)),
        ],
        out_specs=pl.BlockSpec((BLK, D), lambda i: (i, 0)),
        out_shape=jax.ShapeDtypeStruct((N_NODES, D), jnp.float32),
    )(h, parts, deg, w_self, w_neigh)


def kernel(x, edge_index, W_self1, W_neigh1, W_self2, W_neigh2):
    src = edge_index[0].astype(jnp.int32)
    dst = edge_index[1].astype(jnp.int32)
    zeros = jnp.zeros((ROWS_PER_TILE, D), jnp.float32)
    ones = jnp.ones((CHUNK, D), jnp.float32)

    deg_full = _sc_deg(dst, zeros, ones)
    deg = deg_full[:, :, :DW]
    parts1 = _sc_agg(x, src, dst, zeros)
    h1 = _combine(x, parts1, deg, W_self1, W_neigh1, relu=True)
    parts2 = _sc_agg(h1, src, dst, zeros)
    out = _combine(h1, parts2, deg, W_self2, W_neigh2, relu=False)
    return out


# speedup vs baseline: 10.3082x; 2.1641x over previous
import functools

import jax
import jax.numpy as jnp
from jax import lax
from jax.experimental import pallas as pl
from jax.experimental.pallas import tpu as pltpu
from jax.experimental.pallas import tpu_sc as plsc

N_NODES = 10000
N_EDGES = 320000
D = 128

NC = 2
NS = 16
NW = NC * NS
EDGES_PER_WORKER = N_EDGES // NW
CHUNK = 80
NUM_CHUNKS = EDGES_PER_WORKER // CHUNK
N_PAD = 10240
ROWS_PER_TILE = N_PAD // NS
DW = 16

_MESH = plsc.VectorSubcoreMesh(core_axis_name="c", subcore_axis_name="s")


NBUF = 2
LOOP_CHUNKS = (NUM_CHUNKS // NBUF) * NBUF


@functools.partial(
    pl.kernel,
    mesh=_MESH,
    out_type=jax.ShapeDtypeStruct((NC, N_PAD, D), jnp.float32),
    scratch_types=(
        pltpu.VMEM((CHUNK, D), jnp.float32),
        pltpu.VMEM_SHARED((N_PAD, D), jnp.float32),
    )
    + tuple(pltpu.VMEM((CHUNK,), jnp.int32) for _ in range(NBUF))
    + tuple(pltpu.SemaphoreType.DMA for _ in range(NBUF)),
)
def _sc_deg(dst_hbm, zeros_hbm, ones_hbm, deg_out, ones_v, deg_t, *rest):
    dsts = rest[:NBUF]
    dsems = rest[NBUF:]
    c = lax.axis_index("c")
    s = lax.axis_index("s")
    w = c * NS + s
    row0 = s * ROWS_PER_TILE
    base = w * EDGES_PER_WORKER

    pltpu.sync_copy(zeros_hbm, deg_t.at[pl.ds(row0, ROWS_PER_TILE)])
    pltpu.sync_copy(ones_hbm, ones_v)
    plsc.subcore_barrier()

    for b in range(NBUF):
        pltpu.async_copy(dst_hbm.at[pl.ds(base + b * CHUNK, CHUNK)], dsts[b], dsems[b])

    def step(i, b, refill=True):
        pltpu.make_async_copy(dst_hbm.at[pl.ds(base, CHUNK)], dsts[b], dsems[b]).wait()
        pltpu.sync_copy(ones_v, deg_t.at[dsts[b]], add=True)
        if refill:
            nxt = i + NBUF

            @pl.when(nxt < NUM_CHUNKS)
            def _():
                pltpu.async_copy(
                    dst_hbm.at[pl.ds(base + nxt * CHUNK, CHUNK)], dsts[b], dsems[b])

    def body(g, carry):
        for b in range(NBUF):
            step(g * NBUF + b, b)
        return carry

    lax.fori_loop(0, LOOP_CHUNKS // NBUF, body, 0)
    for i in range(LOOP_CHUNKS, NUM_CHUNKS):
        step(i, i % NBUF, refill=False)
    plsc.subcore_barrier()
    pltpu.sync_copy(
        deg_t.at[pl.ds(row0, ROWS_PER_TILE)],
        deg_out.at[c, pl.ds(row0, ROWS_PER_TILE)],
    )


@functools.partial(
    pl.kernel,
    mesh=_MESH,
    out_type=jax.ShapeDtypeStruct((NC, N_PAD, D), jnp.float32),
    scratch_types=(
        pltpu.VMEM((EDGES_PER_WORKER,), jnp.int32),
        pltpu.VMEM_SHARED((N_PAD, D), jnp.float32),
    )
    + tuple(pltpu.VMEM((CHUNK, D), jnp.float32) for _ in range(NBUF))
    + tuple(pltpu.VMEM((CHUNK,), jnp.int32) for _ in range(NBUF))
    + tuple(pltpu.SemaphoreType.DMA for _ in range(2 * NBUF)),
)
def _sc_agg(x_hbm, src_hbm, dst_hbm, zeros_hbm, part_hbm, src_v, acc, *rest):
    rows = rest[:NBUF]
    dsts = rest[NBUF:2 * NBUF]
    gsems = rest[2 * NBUF:3 * NBUF]
    dsems = rest[3 * NBUF:]
    c = lax.axis_index("c")
    s = lax.axis_index("s")
    w = c * NS + s
    row0 = s * ROWS_PER_TILE
    base = w * EDGES_PER_WORKER

    pltpu.sync_copy(src_hbm.at[pl.ds(base, EDGES_PER_WORKER)], src_v)
    pltpu.sync_copy(zeros_hbm, acc.at[pl.ds(row0, ROWS_PER_TILE)])
    plsc.subcore_barrier()

    for b in range(NBUF):
        pltpu.async_copy(x_hbm.at[src_v.at[pl.ds(b * CHUNK, CHUNK)]], rows[b], gsems[b])
        pltpu.async_copy(dst_hbm.at[pl.ds(base + b * CHUNK, CHUNK)], dsts[b], dsems[b])

    def step(i, b, refill=True):
        pltpu.make_async_copy(
            x_hbm.at[src_v.at[pl.ds(0, CHUNK)]], rows[b], gsems[b]).wait()
        pltpu.make_async_copy(
            dst_hbm.at[pl.ds(base, CHUNK)], dsts[b], dsems[b]).wait()
        pltpu.sync_copy(rows[b], acc.at[dsts[b]], add=True)
        if refill:
            nxt = i + NBUF

            @pl.when(nxt < NUM_CHUNKS)
            def _():
                pltpu.async_copy(
                    x_hbm.at[src_v.at[pl.ds(nxt * CHUNK, CHUNK)]], rows[b], gsems[b])
                pltpu.async_copy(
                    dst_hbm.at[pl.ds(base + nxt * CHUNK, CHUNK)], dsts[b], dsems[b])

    def body(g, carry):
        for b in range(NBUF):
            step(g * NBUF + b, b)
        return carry

    lax.fori_loop(0, LOOP_CHUNKS // NBUF, body, 0)
    for i in range(LOOP_CHUNKS, NUM_CHUNKS):
        step(i, i % NBUF, refill=False)
    plsc.subcore_barrier()

    pltpu.sync_copy(
        acc.at[pl.ds(row0, ROWS_PER_TILE)],
        part_hbm.at[c, pl.ds(row0, ROWS_PER_TILE)],
    )


BLK = 1000


def _combine_body(relu):
    def body(h_ref, p_ref, deg_ref, ws_ref, wn_ref, o_ref):
        degs = deg_ref[...]
        deg = degs[0, :, 0] + degs[1, :, 0]
        invd = 1.0 / jnp.maximum(deg, 1.0)
        agg = (p_ref[0] + p_ref[1]) * invd[:, None]
        out = jnp.dot(h_ref[...], ws_ref[...], preferred_element_type=jnp.float32)
        out = out + jnp.dot(agg, wn_ref[...], preferred_element_type=jnp.float32)
        if relu:
            out = jnp.maximum(out, 0.0)
        o_ref[...] = out

    return body


def _combine(h, parts, deg, w_self, w_neigh, relu):
    return pl.pallas_call(
        _combine_body(relu),
        grid=(N_NODES // BLK,),
        in_specs=[
            pl.BlockSpec((BLK, D), lambda i: (i, 0)),
            pl.BlockSpec((NC, BLK, D), lambda i: (0, i, 0)),
            pl.BlockSpec((NC, BLK, DW), lambda i: (0, i, 0)),
            pl.BlockSpec((D, D), lambda i: (0, 0)),
            pl.BlockSpec((D, D), lambda i: (0, 0)),
        ],
        out_specs=pl.BlockSpec((BLK, D), lambda i: (i, 0)),
        out_shape=jax.ShapeDtypeStruct((N_NODES, D), jnp.float32),
    )(h, parts, deg, w_self, w_neigh)


def kernel(x, edge_index, W_self1, W_neigh1, W_self2, W_neigh2):
    src = edge_index[0].astype(jnp.int32)
    dst = edge_index[1].astype(jnp.int32)
    zeros = jnp.zeros((ROWS_PER_TILE, D), jnp.float32)
    ones = jnp.ones((CHUNK, D), jnp.float32)

    deg_full = _sc_deg(dst, zeros, ones)
    deg = deg_full[:, :, :DW]
    parts1 = _sc_agg(x, src, dst, zeros)
    h1 = _combine(x, parts1, deg, W_self1, W_neigh1, relu=True)
    parts2 = _sc_agg(h1, src, dst, zeros)
    out = _combine(h1, parts2, deg, W_self2, W_neigh2, relu=False)
    return out


# speedup vs baseline: 11.8983x; 1.1543x over previous
import functools

import jax
import jax.numpy as jnp
from jax import lax
from jax.experimental import pallas as pl
from jax.experimental.pallas import tpu as pltpu
from jax.experimental.pallas import tpu_sc as plsc

N_NODES = 10000
N_EDGES = 320000
D = 128

NC = 2
NS = 16
NW = NC * NS
EDGES_PER_WORKER = N_EDGES // NW
CHUNK = 80
NUM_CHUNKS = EDGES_PER_WORKER // CHUNK
N_PAD = 10240
ROWS_PER_TILE = N_PAD // NS
DW = 16

_MESH = plsc.VectorSubcoreMesh(core_axis_name="c", subcore_axis_name="s")


NBUF = 3
LOOP_CHUNKS = (NUM_CHUNKS // NBUF) * NBUF


def _make_sc_agg(with_deg: bool):
    out_type = [jax.ShapeDtypeStruct((NC, N_PAD, D), jnp.float32)]
    if with_deg:
        out_type.append(jax.ShapeDtypeStruct((NC, N_PAD, D), jnp.float32))

    @functools.partial(
        pl.kernel,
        mesh=_MESH,
        out_type=tuple(out_type) if with_deg else out_type[0],
        scratch_types=(
            pltpu.VMEM((EDGES_PER_WORKER,), jnp.int32),
            pltpu.VMEM_SHARED((N_PAD, D), jnp.float32),
        )
        + tuple(pltpu.VMEM((CHUNK, D), jnp.float32) for _ in range(NBUF))
        + tuple(pltpu.VMEM((CHUNK,), jnp.int32) for _ in range(NBUF))
        + tuple(pltpu.SemaphoreType.DMA for _ in range(2 * NBUF)),
    )
    def sc_agg(x_hbm, src_hbm, dst_hbm, zeros_hbm, *rest):
        if with_deg:
            ones_hbm, part_hbm, deg_hbm = rest[:3]
            rest = rest[3:]
        else:
            part_hbm = rest[0]
            rest = rest[1:]
        src_v, acc = rest[:2]
        rest = rest[2:]
        rows = rest[:NBUF]
        dsts = rest[NBUF:2 * NBUF]
        gsems = rest[2 * NBUF:3 * NBUF]
        dsems = rest[3 * NBUF:]
        c = lax.axis_index("c")
        s = lax.axis_index("s")
        w = c * NS + s
        row0 = s * ROWS_PER_TILE
        base = w * EDGES_PER_WORKER

        def dst_load(i, b):
            return pltpu.async_copy(
                dst_hbm.at[pl.ds(base + i * CHUNK, CHUNK)], dsts[b], dsems[b])

        def dst_wait(b):
            pltpu.make_async_copy(
                dst_hbm.at[pl.ds(base, CHUNK)], dsts[b], dsems[b]).wait()

        def run_chunks(step):
            def body(g, carry):
                for b in range(NBUF):
                    step(g * NBUF + b, b, True)
                return carry

            lax.fori_loop(0, LOOP_CHUNKS // NBUF, body, 0)
            for i in range(LOOP_CHUNKS, NUM_CHUNKS):
                step(i, i % NBUF, False)

        pltpu.sync_copy(src_hbm.at[pl.ds(base, EDGES_PER_WORKER)], src_v)
        pltpu.sync_copy(zeros_hbm, acc.at[pl.ds(row0, ROWS_PER_TILE)])
        plsc.subcore_barrier()

        if with_deg:
            pltpu.sync_copy(ones_hbm, rows[0])
            for b in range(NBUF):
                dst_load(b, b)

            def deg_step(i, b, refill):
                dst_wait(b)
                pltpu.sync_copy(rows[0], acc.at[dsts[b]], add=True)
                if refill:
                    nxt = i + NBUF

                    @pl.when(nxt < NUM_CHUNKS)
                    def _():
                        dst_load(nxt, b)

            run_chunks(deg_step)
            plsc.subcore_barrier()
            pltpu.sync_copy(
                acc.at[pl.ds(row0, ROWS_PER_TILE)],
                deg_hbm.at[c, pl.ds(row0, ROWS_PER_TILE)],
            )
            pltpu.sync_copy(zeros_hbm, acc.at[pl.ds(row0, ROWS_PER_TILE)])
            plsc.subcore_barrier()

        for b in range(NBUF):
            pltpu.async_copy(
                x_hbm.at[src_v.at[pl.ds(b * CHUNK, CHUNK)]], rows[b], gsems[b])
            dst_load(b, b)

        def agg_step(i, b, refill):
            pltpu.make_async_copy(
                x_hbm.at[src_v.at[pl.ds(0, CHUNK)]], rows[b], gsems[b]).wait()
            dst_wait(b)
            pltpu.sync_copy(rows[b], acc.at[dsts[b]], add=True)
            if refill:
                nxt = i + NBUF

                @pl.when(nxt < NUM_CHUNKS)
                def _():
                    pltpu.async_copy(
                        x_hbm.at[src_v.at[pl.ds(nxt * CHUNK, CHUNK)]],
                        rows[b], gsems[b])
                    dst_load(nxt, b)

        run_chunks(agg_step)
        plsc.subcore_barrier()

        pltpu.sync_copy(
            acc.at[pl.ds(row0, ROWS_PER_TILE)],
            part_hbm.at[c, pl.ds(row0, ROWS_PER_TILE)],
        )

    return sc_agg


_sc_agg_deg = _make_sc_agg(True)
_sc_agg = _make_sc_agg(False)


BLK = 1000


def _combine_body(relu):
    def body(h_ref, p_ref, deg_ref, ws_ref, wn_ref, o_ref):
        degs = deg_ref[...]
        deg = degs[0, :, 0] + degs[1, :, 0]
        invd = 1.0 / jnp.maximum(deg, 1.0)
        agg = (p_ref[0] + p_ref[1]) * invd[:, None]
        out = jnp.dot(h_ref[...], ws_ref[...], preferred_element_type=jnp.float32)
        out = out + jnp.dot(agg, wn_ref[...], preferred_element_type=jnp.float32)
        if relu:
            out = jnp.maximum(out, 0.0)
        o_ref[...] = out

    return body


def _combine(h, parts, deg, w_self, w_neigh, relu):
    return pl.pallas_call(
        _combine_body(relu),
        grid=(N_NODES // BLK,),
        in_specs=[
            pl.BlockSpec((BLK, D), lambda i: (i, 0)),
            pl.BlockSpec((NC, BLK, D), lambda i: (0, i, 0)),
            pl.BlockSpec((NC, BLK, DW), lambda i: (0, i, 0)),
            pl.BlockSpec((D, D), lambda i: (0, 0)),
            pl.BlockSpec((D, D), lambda i: (0, 0)),
        ],
        out_specs=pl.BlockSpec((BLK, D), lambda i: (i, 0)),
        out_shape=jax.ShapeDtypeStruct((N_NODES, D), jnp.float32),
    )(h, parts, deg, w_self, w_neigh)


def kernel(x, edge_index, W_self1, W_neigh1, W_self2, W_neigh2):
    src = edge_index[0].astype(jnp.int32)
    dst = edge_index[1].astype(jnp.int32)
    zeros = jnp.zeros((ROWS_PER_TILE, D), jnp.float32)
    ones = jnp.ones((CHUNK, D), jnp.float32)

    parts1, deg_full = _sc_agg_deg(x, src, dst, zeros, ones)
    deg = deg_full[:, :, :DW]
    h1 = _combine(x, parts1, deg, W_self1, W_neigh1, relu=True)
    parts2 = _sc_agg(h1, src, dst, zeros)
    out = _combine(h1, parts2, deg, W_self2, W_neigh2, relu=False)
    return out


# speedup vs baseline: 12.1733x; 1.0231x over previous
import functools

import jax
import jax.numpy as jnp
from jax import lax
from jax.experimental import pallas as pl
from jax.experimental.pallas import tpu as pltpu
from jax.experimental.pallas import tpu_sc as plsc

N_NODES = 10000
N_EDGES = 320000
D = 128

NC = 2
NS = 16
NW = NC * NS
EDGES_PER_WORKER = N_EDGES // NW
CHUNK = 80
NUM_CHUNKS = EDGES_PER_WORKER // CHUNK
N_PAD = 10240
ROWS_PER_TILE = N_PAD // NS
DW = 16

_MESH = plsc.VectorSubcoreMesh(core_axis_name="c", subcore_axis_name="s")


NBUF = 4
SBUF = 2 * NBUF
LOOP8 = (NUM_CHUNKS // SBUF) * SBUF


def _make_sc_agg(with_deg: bool):
    out_type = [jax.ShapeDtypeStruct((NC, N_PAD, D), jnp.float32)]
    if with_deg:
        out_type.append(jax.ShapeDtypeStruct((NC, N_PAD, D), jnp.float32))

    @functools.partial(
        pl.kernel,
        mesh=_MESH,
        out_type=tuple(out_type) if with_deg else out_type[0],
        scratch_types=(
            pltpu.VMEM_SHARED((N_PAD, D), jnp.float32),
        )
        + tuple(pltpu.VMEM((CHUNK, D), jnp.float32) for _ in range(NBUF))
        + tuple(pltpu.VMEM((CHUNK,), jnp.int32) for _ in range(NBUF))
        + tuple(pltpu.VMEM((CHUNK,), jnp.int32) for _ in range(SBUF))
        + tuple(pltpu.SemaphoreType.DMA for _ in range(2 * NBUF + SBUF)),
    )
    def sc_agg(x_hbm, src_hbm, dst_hbm, zeros_hbm, *rest):
        if with_deg:
            ones_hbm, part_hbm, deg_hbm = rest[:3]
            rest = rest[3:]
        else:
            part_hbm = rest[0]
            rest = rest[1:]
        acc = rest[0]
        rest = rest[1:]
        rows = rest[:NBUF]
        dsts = rest[NBUF:2 * NBUF]
        srcs = rest[2 * NBUF:2 * NBUF + SBUF]
        gsems = rest[2 * NBUF + SBUF:3 * NBUF + SBUF]
        dsems = rest[3 * NBUF + SBUF:4 * NBUF + SBUF]
        ssems = rest[4 * NBUF + SBUF:]
        c = lax.axis_index("c")
        s = lax.axis_index("s")
        w = c * NS + s
        row0 = s * ROWS_PER_TILE
        base = w * EDGES_PER_WORKER

        def dst_load(i, b):
            pltpu.async_copy(
                dst_hbm.at[pl.ds(base + i * CHUNK, CHUNK)], dsts[b], dsems[b])

        def dst_wait(b):
            pltpu.make_async_copy(
                dst_hbm.at[pl.ds(base, CHUNK)], dsts[b], dsems[b]).wait()

        def src_load(i, q):
            pltpu.async_copy(
                src_hbm.at[pl.ds(base + i * CHUNK, CHUNK)], srcs[q], ssems[q])

        def src_wait(q):
            pltpu.make_async_copy(
                src_hbm.at[pl.ds(base, CHUNK)], srcs[q], ssems[q]).wait()

        def gather(i_q):
            pltpu.async_copy(x_hbm.at[srcs[i_q % SBUF]], rows[i_q % NBUF],
                             gsems[i_q % NBUF])

        pltpu.sync_copy(zeros_hbm, acc.at[pl.ds(row0, ROWS_PER_TILE)])
        plsc.subcore_barrier()

        if with_deg:
            pltpu.sync_copy(ones_hbm, rows[0])
            for b in range(NBUF):
                dst_load(b, b)

            def deg_step(i, b, refill):
                dst_wait(b)
                pltpu.sync_copy(rows[0], acc.at[dsts[b]], add=True)
                if refill:
                    nxt = i + NBUF

                    @pl.when(nxt < NUM_CHUNKS)
                    def _():
                        dst_load(nxt, b)

            def deg_body(g, carry):
                for b in range(NBUF):
                    deg_step(g * NBUF + b, b, True)
                return carry

            n_full = (NUM_CHUNKS // NBUF) * NBUF
            lax.fori_loop(0, n_full // NBUF, deg_body, 0)
            for i in range(n_full, NUM_CHUNKS):
                deg_step(i, i % NBUF, False)
            plsc.subcore_barrier()
            pltpu.sync_copy(
                acc.at[pl.ds(row0, ROWS_PER_TILE)],
                deg_hbm.at[c, pl.ds(row0, ROWS_PER_TILE)],
            )
            pltpu.sync_copy(zeros_hbm, acc.at[pl.ds(row0, ROWS_PER_TILE)])
            plsc.subcore_barrier()

        for q in range(SBUF):
            src_load(q, q)
        for b in range(NBUF):
            src_wait(b)
            gather(b)
            dst_load(b, b)

        def agg_body(g, carry):
            for k in range(SBUF):
                i = g * SBUF + k
                b = k % NBUF
                q = k
                pltpu.make_async_copy(
                    x_hbm.at[srcs[q]], rows[b], gsems[b]).wait()
                dst_wait(b)
                pltpu.sync_copy(rows[b], acc.at[dsts[b]], add=True)
                nxt = i + NBUF
                nq = (k + NBUF) % SBUF
                src_wait(nq)
                pltpu.async_copy(x_hbm.at[srcs[nq]], rows[b], gsems[b])
                dst_load(nxt, b)
                nxt2 = i + SBUF

                @pl.when(nxt2 < NUM_CHUNKS)
                def _():
                    src_load(nxt2, q)
            return carry

        lax.fori_loop(0, LOOP8 // SBUF, agg_body, 0)
        for i in range(LOOP8, NUM_CHUNKS):
            b = i % NBUF
            q = i % SBUF
            pltpu.make_async_copy(x_hbm.at[srcs[q]], rows[b], gsems[b]).wait()
            dst_wait(b)
            pltpu.sync_copy(rows[b], acc.at[dsts[b]], add=True)
            nxt = i + NBUF
            if nxt < NUM_CHUNKS:
                nq = nxt % SBUF
                src_wait(nq)
                pltpu.async_copy(x_hbm.at[srcs[nq]], rows[b], gsems[b])
                dst_load(nxt, b)
        plsc.subcore_barrier()

        pltpu.sync_copy(
            acc.at[pl.ds(row0, ROWS_PER_TILE)],
            part_hbm.at[c, pl.ds(row0, ROWS_PER_TILE)],
        )

    return sc_agg


_sc_agg_deg = _make_sc_agg(True)
_sc_agg = _make_sc_agg(False)


BLK = 1000


def _combine_body(relu):
    def body(h_ref, p_ref, deg_ref, ws_ref, wn_ref, o_ref):
        degs = deg_ref[...]
        deg = degs[0, :, 0] + degs[1, :, 0]
        invd = 1.0 / jnp.maximum(deg, 1.0)
        agg = (p_ref[0] + p_ref[1]) * invd[:, None]
        out = jnp.dot(h_ref[...], ws_ref[...], preferred_element_type=jnp.float32)
        out = out + jnp.dot(agg, wn_ref[...], preferred_element_type=jnp.float32)
        if relu:
            out = jnp.maximum(out, 0.0)
        o_ref[...] = out

    return body


def _combine(h, parts, deg, w_self, w_neigh, relu):
    return pl.pallas_call(
        _combine_body(relu),
        grid=(N_NODES // BLK,),
        in_specs=[
            pl.BlockSpec((BLK, D), lambda i: (i, 0)),
            pl.BlockSpec((NC, BLK, D), lambda i: (0, i, 0)),
            pl.BlockSpec((NC, BLK, D), lambda i: (0, i, 0)),
            pl.BlockSpec((D, D), lambda i: (0, 0)),
            pl.BlockSpec((D, D), lambda i: (0, 0)),
        ],
        out_specs=pl.BlockSpec((BLK, D), lambda i: (i, 0)),
        out_shape=jax.ShapeDtypeStruct((N_NODES, D), jnp.float32),
    )(h, parts, deg, w_self, w_neigh)


def kernel(x, edge_index, W_self1, W_neigh1, W_self2, W_neigh2):
    src = edge_index[0].astype(jnp.int32)
    dst = edge_index[1].astype(jnp.int32)
    zeros = jnp.zeros((ROWS_PER_TILE, D), jnp.float32)
    ones = jnp.ones((CHUNK, D), jnp.float32)

    parts1, deg = _sc_agg_deg(x, src, dst, zeros, ones)
    h1 = _combine(x, parts1, deg, W_self1, W_neigh1, relu=True)
    parts2 = _sc_agg(h1, src, dst, zeros)
    out = _combine(h1, parts2, deg, W_self2, W_neigh2, relu=False)
    return out


# speedup vs baseline: 12.6771x; 1.0414x over previous
import functools

import jax
import jax.numpy as jnp
from jax import lax
from jax.experimental import pallas as pl
from jax.experimental.pallas import tpu as pltpu
from jax.experimental.pallas import tpu_sc as plsc

N_NODES = 10000
N_EDGES = 320000
D = 128

NC = 2
NS = 16
NW = NC * NS
EDGES_PER_WORKER = N_EDGES // NW
CHUNK = 80
NUM_CHUNKS = EDGES_PER_WORKER // CHUNK
N_PAD = 10240
ROWS_PER_TILE = N_PAD // NS
DW = 16

_MESH = plsc.VectorSubcoreMesh(core_axis_name="c", subcore_axis_name="s")


NBUF = 4
SBUF = 2 * NBUF
LOOP8 = (NUM_CHUNKS // SBUF) * SBUF


def _make_sc_agg(with_deg: bool):
    out_type = [jax.ShapeDtypeStruct((NC, N_PAD, D), jnp.float32)]
    if with_deg:
        out_type.append(jax.ShapeDtypeStruct((NC, N_PAD, D), jnp.float32))

    @functools.partial(
        pl.kernel,
        mesh=_MESH,
        out_type=tuple(out_type) if with_deg else out_type[0],
        scratch_types=(
            pltpu.VMEM_SHARED((N_PAD, D), jnp.float32),
        )
        + tuple(pltpu.VMEM((CHUNK, D), jnp.float32) for _ in range(NBUF))
        + tuple(pltpu.VMEM((CHUNK,), jnp.int32) for _ in range(NBUF))
        + tuple(pltpu.VMEM((CHUNK,), jnp.int32) for _ in range(SBUF))
        + tuple(pltpu.SemaphoreType.DMA for _ in range(2 * NBUF + SBUF)),
    )
    def sc_agg(x_hbm, src_hbm, dst_hbm, *rest):
        if with_deg:
            part_hbm, deg_hbm = rest[:2]
            rest = rest[2:]
        else:
            part_hbm = rest[0]
            rest = rest[1:]
        acc = rest[0]
        rest = rest[1:]
        rows = rest[:NBUF]
        dsts = rest[NBUF:2 * NBUF]
        srcs = rest[2 * NBUF:2 * NBUF + SBUF]
        gsems = rest[2 * NBUF + SBUF:3 * NBUF + SBUF]
        dsems = rest[3 * NBUF + SBUF:4 * NBUF + SBUF]
        ssems = rest[4 * NBUF + SBUF:]
        c = lax.axis_index("c")
        s = lax.axis_index("s")
        w = c * NS + s
        row0 = s * ROWS_PER_TILE
        base = w * EDGES_PER_WORKER

        def dst_load(i, b):
            pltpu.async_copy(
                dst_hbm.at[pl.ds(base + i * CHUNK, CHUNK)], dsts[b], dsems[b])

        def dst_wait(b):
            pltpu.make_async_copy(
                dst_hbm.at[pl.ds(base, CHUNK)], dsts[b], dsems[b]).wait()

        def src_load(i, q):
            pltpu.async_copy(
                src_hbm.at[pl.ds(base + i * CHUNK, CHUNK)], srcs[q], ssems[q])

        def src_wait(q):
            pltpu.make_async_copy(
                src_hbm.at[pl.ds(base, CHUNK)], srcs[q], ssems[q]).wait()

        def gather(i_q):
            pltpu.async_copy(x_hbm.at[srcs[i_q % SBUF]], rows[i_q % NBUF],
                             gsems[i_q % NBUF])

        z16 = jnp.zeros((16,), jnp.float32)

        def fill_row(r, carry):
            for j in range(D // 16):
                rows[1][r, pl.ds(j * 16, 16)] = z16
            return carry

        lax.fori_loop(0, CHUNK, fill_row, 0)

        def zero_acc_slice():
            for j in range(ROWS_PER_TILE // CHUNK):
                pltpu.sync_copy(
                    rows[1], acc.at[pl.ds(row0 + j * CHUNK, CHUNK)])

        zero_acc_slice()
        plsc.subcore_barrier()

        if with_deg:
            o16 = jnp.ones((16,), jnp.float32)

            def fill_ones(r, carry):
                for j in range(D // 16):
                    rows[0][r, pl.ds(j * 16, 16)] = o16
                return carry

            lax.fori_loop(0, CHUNK, fill_ones, 0)
            for b in range(NBUF):
                dst_load(b, b)

            def deg_step(i, b, refill):
                dst_wait(b)
                pltpu.sync_copy(rows[0], acc.at[dsts[b]], add=True)
                if refill:
                    nxt = i + NBUF

                    @pl.when(nxt < NUM_CHUNKS)
                    def _():
                        dst_load(nxt, b)

            def deg_body(g, carry):
                for b in range(NBUF):
                    deg_step(g * NBUF + b, b, True)
                return carry

            n_full = (NUM_CHUNKS // NBUF) * NBUF
            lax.fori_loop(0, n_full // NBUF, deg_body, 0)
            for i in range(n_full, NUM_CHUNKS):
                deg_step(i, i % NBUF, False)
            plsc.subcore_barrier()
            pltpu.sync_copy(
                acc.at[pl.ds(row0, ROWS_PER_TILE)],
                deg_hbm.at[c, pl.ds(row0, ROWS_PER_TILE)],
            )
            zero_acc_slice()
            plsc.subcore_barrier()

        for q in range(SBUF):
            src_load(q, q)
        for b in range(NBUF):
            src_wait(b)
            gather(b)
            dst_load(b, b)

        def agg_body(g, carry):
            for k in range(SBUF):
                i = g * SBUF + k
                b = k % NBUF
                q = k
                pltpu.make_async_copy(
                    x_hbm.at[srcs[q]], rows[b], gsems[b]).wait()
                dst_wait(b)
                pltpu.sync_copy(rows[b], acc.at[dsts[b]], add=True)
                nxt = i + NBUF
                nq = (k + NBUF) % SBUF
                src_wait(nq)
                pltpu.async_copy(x_hbm.at[srcs[nq]], rows[b], gsems[b])
                dst_load(nxt, b)
                nxt2 = i + SBUF

                @pl.when(nxt2 < NUM_CHUNKS)
                def _():
                    src_load(nxt2, q)
            return carry

        lax.fori_loop(0, LOOP8 // SBUF, agg_body, 0)
        for i in range(LOOP8, NUM_CHUNKS):
            b = i % NBUF
            q = i % SBUF
            pltpu.make_async_copy(x_hbm.at[srcs[q]], rows[b], gsems[b]).wait()
            dst_wait(b)
            pltpu.sync_copy(rows[b], acc.at[dsts[b]], add=True)
            nxt = i + NBUF
            if nxt < NUM_CHUNKS:
                nq = nxt % SBUF
                src_wait(nq)
                pltpu.async_copy(x_hbm.at[srcs[nq]], rows[b], gsems[b])
                dst_load(nxt, b)
        plsc.subcore_barrier()

        pltpu.sync_copy(
            acc.at[pl.ds(row0, ROWS_PER_TILE)],
            part_hbm.at[c, pl.ds(row0, ROWS_PER_TILE)],
        )

    return sc_agg


_sc_agg_deg = _make_sc_agg(True)
_sc_agg = _make_sc_agg(False)


BLK = 1000


def _combine_body(relu):
    def body(h_ref, p_ref, deg_ref, ws_ref, wn_ref, o_ref):
        degs = deg_ref[...]
        deg = degs[0, :, 0] + degs[1, :, 0]
        invd = 1.0 / jnp.maximum(deg, 1.0)
        agg = (p_ref[0] + p_ref[1]) * invd[:, None]
        out = jnp.dot(h_ref[...], ws_ref[...], preferred_element_type=jnp.float32)
        out = out + jnp.dot(agg, wn_ref[...], preferred_element_type=jnp.float32)
        if relu:
            out = jnp.maximum(out, 0.0)
        o_ref[...] = out

    return body


def _combine(h, parts, deg, w_self, w_neigh, relu):
    return pl.pallas_call(
        _combine_body(relu),
        grid=(N_NODES // BLK,),
        in_specs=[
            pl.BlockSpec((BLK, D), lambda i: (i, 0)),
            pl.BlockSpec((NC, BLK, D), lambda i: (0, i, 0)),
            pl.BlockSpec((NC, BLK, D), lambda i: (0, i, 0)),
            pl.BlockSpec((D, D), lambda i: (0, 0)),
            pl.BlockSpec((D, D), lambda i: (0, 0)),
        ],
        out_specs=pl.BlockSpec((BLK, D), lambda i: (i, 0)),
        out_shape=jax.ShapeDtypeStruct((N_NODES, D), jnp.float32),
    )(h, parts, deg, w_self, w_neigh)


def kernel(x, edge_index, W_self1, W_neigh1, W_self2, W_neigh2):
    src = edge_index[0].astype(jnp.int32)
    dst = edge_index[1].astype(jnp.int32)

    parts1, deg = _sc_agg_deg(x, src, dst)
    h1 = _combine(x, parts1, deg, W_self1, W_neigh1, relu=True)
    parts2 = _sc_agg(h1, src, dst)
    out = _combine(h1, parts2, deg, W_self2, W_neigh2, relu=False)
    return out
